# Initial kernel scaffold; baseline (speedup 1.0000x reference)
import jax
import jax.numpy as jnp
from jax.experimental import pallas as pl


def kernel(x, edge_index, edge_attr, batch, W1, b1, Wq, bq, Wk, bk, Wv, bv, Wo, bo, W2, b2):
    raise NotImplementedError("write your pallas kernel here")


# speedup vs baseline: 27.0152x; 27.0152x over previous
import functools

import jax
import jax.numpy as jnp
from jax import lax
from jax.experimental import pallas as pl
from jax.experimental.pallas import tpu as pltpu
from jax.experimental.pallas import tpu_sc as plsc

_N = 10000
_NP = 10240
_E = 320000
_NC = 2
_NS = 16
_NW = _NC * _NS
_EPT = _E // _NW
_CH = 80
_NCH = _EPT // _CH
_NPT = _NP // _NS
_HID = 64
_LANES = 16


def _mesh():
    return plsc.VectorSubcoreMesh(
        core_axis_name="c", subcore_axis_name="s",
        num_cores=_NC, num_subcores=_NS)


def _splat_lane(vec, lane):
    dn = lax.GatherDimensionNumbers(
        offset_dims=(), collapsed_slice_dims=(0,), start_index_map=(0,))
    idx = jnp.full((_LANES, 1), lane, jnp.int32)
    return lax.gather(vec, idx, dn, slice_sizes=(1,),
                      mode=lax.GatherScatterMode.PROMISE_IN_BOUNDS)


@functools.partial(
    pl.kernel,
    out_type=jax.ShapeDtypeStruct((_NC, _NP), jnp.float32),
    mesh=_mesh(),
    scratch_types=[
        pltpu.VMEM((_NCH, _CH), jnp.int32),
        pltpu.VMEM((_EPT,), jnp.float32),
        pltpu.VMEM((_NPT,), jnp.float32),
        pltpu.VMEM_SHARED((_NP,), jnp.float32),
    ],
)
def _sc_deg(col2_hbm, ew_hbm, out_hbm, colv, ewv, zb, acc):
    cid = lax.axis_index("c")
    sid = lax.axis_index("s")
    wid = cid * _NS + sid
    zero16 = jnp.zeros((_LANES,), jnp.float32)

    def zfill(i, _):
        zb[pl.ds(i * _LANES, _LANES)] = zero16
        return 0
    lax.fori_loop(0, _NPT // _LANES, zfill, 0)
    pltpu.sync_copy(zb, acc.at[pl.ds(sid * _NPT, _NPT)])
    plsc.subcore_barrier()

    pltpu.sync_copy(col2_hbm.at[wid], colv)
    pltpu.sync_copy(ew_hbm.at[wid], ewv)

    def chunk(i, _):
        pltpu.sync_copy(ewv.at[pl.ds(i * _CH, _CH)], acc.at[colv.at[i]],
                        add=True)
        return 0
    lax.fori_loop(0, _NCH, chunk, 0)
    plsc.subcore_barrier()
    pltpu.sync_copy(acc.at[pl.ds(sid * _NPT, _NPT)],
                    out_hbm.at[cid, pl.ds(sid * _NPT, _NPT)])


@functools.partial(
    pl.kernel,
    out_type=(jax.ShapeDtypeStruct((_NC, _NP, _HID), jnp.float32),
              jax.ShapeDtypeStruct((_NC, _NP), jnp.float32)),
    mesh=_mesh(),
    compiler_params=pltpu.CompilerParams(use_tc_tiling_on_sc=False),
    scratch_types=[
        pltpu.VMEM((_NCH, _CH), jnp.int32),
        pltpu.VMEM((_NCH, _CH), jnp.int32),
        pltpu.VMEM((_EPT,), jnp.float32),
        pltpu.VMEM((_CH,), jnp.float32),
        pltpu.VMEM((_EPT,), jnp.float32),
        pltpu.VMEM((_CH, _HID), jnp.float32),
        pltpu.VMEM((_NPT,), jnp.float32),
        pltpu.VMEM_SHARED((_NP, _HID), jnp.float32),
        pltpu.VMEM_SHARED((_NP,), jnp.float32),
    ],
)
def _sc_agg(row2_hbm, col2_hbm, ew_hbm, dinv_hbm, xws_hbm,
            feat_hbm, cred_hbm,
            row2v, col2v, ewv, dval, cvalv, rows_a, zb,
            spfeat, spcred):
    cid = lax.axis_index("c")
    sid = lax.axis_index("s")
    wid = cid * _NS + sid
    zero16 = jnp.zeros((_LANES,), jnp.float32)

    def zrow(i, _):
        for k in range(_HID // _LANES):
            rows_a[i, pl.ds(k * _LANES, _LANES)] = zero16
        return 0
    lax.fori_loop(0, _CH, zrow, 0)
    for j in range(_NPT // _CH):
        pltpu.sync_copy(rows_a,
                        spfeat.at[pl.ds(sid * _NPT + j * _CH, _CH)])

    def zfill(i, _):
        zb[pl.ds(i * _LANES, _LANES)] = zero16
        return 0
    lax.fori_loop(0, _NPT // _LANES, zfill, 0)
    pltpu.sync_copy(zb, spcred.at[pl.ds(sid * _NPT, _NPT)])
    plsc.subcore_barrier()

    pltpu.sync_copy(row2_hbm.at[wid], row2v)
    pltpu.sync_copy(col2_hbm.at[wid], col2v)
    pltpu.sync_copy(ew_hbm.at[wid], ewv)

    def chunk(i, _):
        pltpu.sync_copy(xws_hbm.at[row2v.at[i]], rows_a)

        for g in range(_CH // _LANES):
            wv = ewv[pl.ds(i * _CH + g * _LANES, _LANES)]
            for e in range(_LANES):
                ws = _splat_lane(wv, e)
                r = g * _LANES + e
                for k in range(_HID // _LANES):
                    sl = pl.ds(k * _LANES, _LANES)
                    rows_a[r, sl] = rows_a[r, sl] * ws

        pltpu.sync_copy(rows_a, spfeat.at[col2v.at[i]], add=True)
        return 0
    lax.fori_loop(0, _NCH, chunk, 0)

    def cgrp(i, _):
        pltpu.sync_copy(dinv_hbm.at[col2v.at[i]], dval)
        for g in range(_CH // _LANES):
            sl = pl.ds(i * _CH + g * _LANES, _LANES)
            gsl = pl.ds(g * _LANES, _LANES)
            cvalv[sl] = ewv[sl] * dval[gsl]
        return 0
    lax.fori_loop(0, _NCH, cgrp, 0)

    def cscat(i, _):
        pltpu.sync_copy(cvalv.at[pl.ds(i * _CH, _CH)],
                        spcred.at[row2v.at[i]], add=True)
        return 0
    lax.fori_loop(0, _NCH, cscat, 0)

    plsc.subcore_barrier()
    pltpu.sync_copy(spfeat.at[pl.ds(sid * _NPT, _NPT)],
                    feat_hbm.at[cid, pl.ds(sid * _NPT, _NPT)])
    pltpu.sync_copy(spcred.at[pl.ds(sid * _NPT, _NPT)],
                    cred_hbm.at[cid, pl.ds(sid * _NPT, _NPT)])


def _tc_pre_body(x_ref, w1_ref, degp_ref, xws_ref, dinv_ref):
    deg = degp_ref[:, 0:1] + degp_ref[:, 1:2] + 1.0
    dinv = lax.rsqrt(deg)
    xw = jnp.dot(x_ref[:, :], w1_ref[:, :], preferred_element_type=jnp.float32)
    xws_ref[:, :] = xw * dinv
    dinv_ref[:, :] = dinv


def _tc_post_body(feat_ref, credp_ref, xws_ref, dinv_ref, b1_ref,
                  wv_ref, bv_ref, wo_ref, bo_ref, w2_ref, b2_ref, out_ref):
    agg = feat_ref[0] + feat_ref[1]
    dinv = dinv_ref[:, :]
    h = jnp.maximum(dinv * (agg + xws_ref[:, :]) + b1_ref[:, :], 0.0)
    t = jnp.dot(h, wv_ref[:, :], preferred_element_type=jnp.float32) + bv_ref[:, :]
    a = jnp.dot(t, wo_ref[:, :], preferred_element_type=jnp.float32) + bo_ref[:, :]
    cred = credp_ref[:, 0:1] + credp_ref[:, 1:2]
    c = dinv * (cred + dinv)
    ridx = lax.broadcasted_iota(jnp.int32, (_NP, 1), 0)
    c = jnp.where(ridx < _N, c, 0.0)
    s = jnp.sum(c * a, axis=0, keepdims=True)
    out_ref[:, :] = (jnp.dot(s, w2_ref[:, :], preferred_element_type=jnp.float32)
                     * (1.0 / _N) + b2_ref[:, :])


def kernel(x, edge_index, edge_attr, batch, W1, b1, Wq, bq, Wk, bk,
           Wv, bv, Wo, bo, W2, b2):
    del batch, Wq, bq, Wk, bk
    row = edge_index[0]
    col = edge_index[1]
    row2 = row.reshape(_NW, _NCH, _CH)
    col2 = col.reshape(_NW, _NCH, _CH)
    ew2 = edge_attr.reshape(_NW, _EPT)
    xpad = jnp.pad(x, ((0, _NP - _N), (0, 0)))

    degp = _sc_deg(col2, ew2)

    xws, dinv2 = pl.pallas_call(
        _tc_pre_body,
        out_shape=(jax.ShapeDtypeStruct((_NP, _HID), jnp.float32),
                   jax.ShapeDtypeStruct((_NP, 1), jnp.float32)),
    )(xpad, W1, degp.T)

    dinv1 = dinv2[:_N, 0]
    feat, credp = _sc_agg(row2, col2, ew2, dinv1, xws)

    out = pl.pallas_call(
        _tc_post_body,
        out_shape=jax.ShapeDtypeStruct((1, 87), jnp.float32),
    )(feat, credp.T, xws, dinv2, b1[None], Wv, bv[None], Wo, bo[None],
      W2, b2[None])
    return out


# speedup vs baseline: 32.7114x; 1.2109x over previous
import functools

import jax
import jax.numpy as jnp
from jax import lax
from jax.experimental import pallas as pl
from jax.experimental.pallas import tpu as pltpu
from jax.experimental.pallas import tpu_sc as plsc

_N = 10000
_NP = 10240
_E = 320000
_NC = 2
_NS = 16
_NW = _NC * _NS
_EPT = _E // _NW
_CH = 80
_NCH = _EPT // _CH
_SUB = 5
_MCH = _CH * _SUB
_NMC = _EPT // _MCH
_NPT = _NP // _NS
_HID = 64
_LANES = 16


def _mesh():
    return plsc.VectorSubcoreMesh(
        core_axis_name="c", subcore_axis_name="s",
        num_cores=_NC, num_subcores=_NS)


def _splat_lane(vec, lane):
    dn = lax.GatherDimensionNumbers(
        offset_dims=(), collapsed_slice_dims=(0,), start_index_map=(0,))
    idx = jnp.full((_LANES, 1), lane, jnp.int32)
    return lax.gather(vec, idx, dn, slice_sizes=(1,),
                      mode=lax.GatherScatterMode.PROMISE_IN_BOUNDS)


@functools.partial(
    pl.kernel,
    out_type=jax.ShapeDtypeStruct((_NC, _NP), jnp.float32),
    mesh=_mesh(),
    scratch_types=[
        pltpu.VMEM((_NCH, _CH), jnp.int32),
        pltpu.VMEM((_EPT,), jnp.float32),
        pltpu.VMEM((_NPT,), jnp.float32),
        pltpu.VMEM_SHARED((_NP,), jnp.float32),
    ],
)
def _sc_deg(col2_hbm, ew_hbm, out_hbm, colv, ewv, zb, acc):
    cid = lax.axis_index("c")
    sid = lax.axis_index("s")
    wid = cid * _NS + sid
    zero16 = jnp.zeros((_LANES,), jnp.float32)

    def zfill(i, _):
        zb[pl.ds(i * _LANES, _LANES)] = zero16
        return 0
    lax.fori_loop(0, _NPT // _LANES, zfill, 0)
    pltpu.sync_copy(zb, acc.at[pl.ds(sid * _NPT, _NPT)])
    plsc.subcore_barrier()

    pltpu.sync_copy(col2_hbm.at[wid], colv)
    pltpu.sync_copy(ew_hbm.at[wid], ewv)

    def chunk(i, _):
        pltpu.sync_copy(ewv.at[pl.ds(i * _CH, _CH)], acc.at[colv.at[i]],
                        add=True)
        return 0
    lax.fori_loop(0, _NCH, chunk, 0)
    plsc.subcore_barrier()
    pltpu.sync_copy(acc.at[pl.ds(sid * _NPT, _NPT)],
                    out_hbm.at[cid, pl.ds(sid * _NPT, _NPT)])


@functools.partial(
    pl.kernel,
    out_type=(jax.ShapeDtypeStruct((_NC, _NP, _HID), jnp.float32),
              jax.ShapeDtypeStruct((_NC, _NP), jnp.float32)),
    mesh=_mesh(),
    compiler_params=pltpu.CompilerParams(use_tc_tiling_on_sc=False),
    scratch_types=[
        pltpu.VMEM((_NCH, _CH), jnp.int32),
        pltpu.VMEM((_NCH, _CH), jnp.int32),
        pltpu.VMEM((_EPT,), jnp.float32),
        pltpu.VMEM((2, _MCH), jnp.float32),
        pltpu.VMEM((2, _MCH), jnp.float32),
        pltpu.VMEM((_MCH, _HID), jnp.float32),
        pltpu.VMEM((_MCH, _HID), jnp.float32),
        pltpu.VMEM((_NPT,), jnp.float32),
        pltpu.VMEM_SHARED((_NP, _HID), jnp.float32),
        pltpu.VMEM_SHARED((_NP,), jnp.float32),
        pltpu.SemaphoreType.DMA,
        pltpu.SemaphoreType.DMA,
        pltpu.SemaphoreType.DMA,
        pltpu.SemaphoreType.DMA,
        pltpu.SemaphoreType.DMA,
        pltpu.SemaphoreType.DMA,
        pltpu.SemaphoreType.DMA,
        pltpu.SemaphoreType.DMA,
    ],
)
def _sc_agg(row2_hbm, col2_hbm, ew_hbm, dinv_hbm, xws_hbm,
            feat_hbm, cred_hbm,
            row2v, col2v, ewv, dval, cval, rows_a, rows_b, zb,
            spfeat, spcred,
            gsem_a, gsem_b, ssem_a, ssem_b, dsem_a, dsem_b, csem_a, csem_b):
    cid = lax.axis_index("c")
    sid = lax.axis_index("s")
    wid = cid * _NS + sid
    zero16 = jnp.zeros((_LANES,), jnp.float32)

    def zrow(i, _):
        for k in range(_HID // _LANES):
            rows_a[i, pl.ds(k * _LANES, _LANES)] = zero16
        return 0
    lax.fori_loop(0, _MCH, zrow, 0)
    for j in range(_NPT // _MCH + 1):
        n = min(_NPT - j * _MCH, _MCH)
        pltpu.sync_copy(rows_a.at[pl.ds(0, n)],
                        spfeat.at[pl.ds(sid * _NPT + j * _MCH, n)])

    def zfill(i, _):
        zb[pl.ds(i * _LANES, _LANES)] = zero16
        return 0
    lax.fori_loop(0, _NPT // _LANES, zfill, 0)
    pltpu.sync_copy(zb, spcred.at[pl.ds(sid * _NPT, _NPT)])
    plsc.subcore_barrier()

    pltpu.sync_copy(row2_hbm.at[wid], row2v)
    pltpu.sync_copy(col2_hbm.at[wid], col2v)
    pltpu.sync_copy(ew_hbm.at[wid], ewv)

    bufs = (rows_a, rows_b)
    gsems = (gsem_a, gsem_b)
    ssems = (ssem_a, ssem_b)
    dsems = (dsem_a, dsem_b)
    csems = (csem_a, csem_b)

    def issue_gather(i, p):
        for j in range(_SUB):
            pltpu.async_copy(xws_hbm.at[row2v.at[i * _SUB + j]],
                             bufs[p].at[pl.ds(j * _CH, _CH)], gsems[p])
            pltpu.async_copy(dinv_hbm.at[col2v.at[i * _SUB + j]],
                             dval.at[p, pl.ds(j * _CH, _CH)], dsems[p])

    def wait_gather(i, p):
        for j in range(_SUB):
            pltpu.make_async_copy(xws_hbm.at[row2v.at[i * _SUB + j]],
                                  bufs[p].at[pl.ds(j * _CH, _CH)],
                                  gsems[p]).wait()
            pltpu.make_async_copy(dinv_hbm.at[col2v.at[i * _SUB + j]],
                                  dval.at[p, pl.ds(j * _CH, _CH)],
                                  dsems[p]).wait()

    def scale(i, p):
        def grp(g, _):
            wv = ewv[pl.ds(i * _MCH + g * _LANES, _LANES)]
            dv = dval[p, pl.ds(g * _LANES, _LANES)]
            cval[p, pl.ds(g * _LANES, _LANES)] = wv * dv
            for e in range(_LANES):
                ws = _splat_lane(wv, e)
                r = g * _LANES + e
                for k in range(_HID // _LANES):
                    sl = pl.ds(k * _LANES, _LANES)
                    bufs[p][r, sl] = bufs[p][r, sl] * ws
            return 0
        lax.fori_loop(0, _MCH // _LANES, grp, 0)

    def issue_scatter(i, p):
        for j in range(_SUB):
            pltpu.async_copy(bufs[p].at[pl.ds(j * _CH, _CH)],
                             spfeat.at[col2v.at[i * _SUB + j]], ssems[p],
                             add=True)
            pltpu.async_copy(cval.at[p, pl.ds(j * _CH, _CH)],
                             spcred.at[row2v.at[i * _SUB + j]], csems[p],
                             add=True)

    def wait_scatter(i, p):
        for j in range(_SUB):
            pltpu.make_async_copy(bufs[p].at[pl.ds(j * _CH, _CH)],
                                  spfeat.at[col2v.at[i * _SUB + j]],
                                  ssems[p]).wait()
            pltpu.make_async_copy(cval.at[p, pl.ds(j * _CH, _CH)],
                                  spcred.at[row2v.at[i * _SUB + j]],
                                  csems[p]).wait()

    issue_gather(0, 0)

    def pair(t, _):
        i0 = 2 * t
        @pl.when(t > 0)
        def _():
            wait_scatter(i0 - 1, 1)
        issue_gather(i0 + 1, 1)
        wait_gather(i0, 0)
        scale(i0, 0)
        issue_scatter(i0, 0)
        wait_gather(i0 + 1, 1)
        scale(i0 + 1, 1)
        wait_scatter(i0, 0)
        issue_gather(i0 + 2, 0)
        issue_scatter(i0 + 1, 1)
        return 0
    lax.fori_loop(0, (_NMC - 1) // 2, pair, 0)

    last = _NMC - 1
    wait_scatter(last - 1, 1)
    wait_gather(last, 0)
    scale(last, 0)
    issue_scatter(last, 0)
    wait_scatter(last, 0)

    plsc.subcore_barrier()
    pltpu.sync_copy(spfeat.at[pl.ds(sid * _NPT, _NPT)],
                    feat_hbm.at[cid, pl.ds(sid * _NPT, _NPT)])
    pltpu.sync_copy(spcred.at[pl.ds(sid * _NPT, _NPT)],
                    cred_hbm.at[cid, pl.ds(sid * _NPT, _NPT)])


def _tc_pre_body(x_ref, w1_ref, degp_ref, xws_ref, dinv_ref):
    deg = degp_ref[:, 0:1] + degp_ref[:, 1:2] + 1.0
    dinv = lax.rsqrt(deg)
    xw = jnp.dot(x_ref[:, :], w1_ref[:, :], preferred_element_type=jnp.float32)
    xws_ref[:, :] = xw * dinv
    dinv_ref[:, :] = dinv


def _tc_post_body(feat_ref, credp_ref, xws_ref, dinv_ref, b1_ref,
                  wv_ref, bv_ref, wo_ref, bo_ref, w2_ref, b2_ref, out_ref):
    agg = feat_ref[0] + feat_ref[1]
    dinv = dinv_ref[:, :]
    h = jnp.maximum(dinv * (agg + xws_ref[:, :]) + b1_ref[:, :], 0.0)
    t = jnp.dot(h, wv_ref[:, :], preferred_element_type=jnp.float32) + bv_ref[:, :]
    a = jnp.dot(t, wo_ref[:, :], preferred_element_type=jnp.float32) + bo_ref[:, :]
    cred = credp_ref[:, 0:1] + credp_ref[:, 1:2]
    c = dinv * (cred + dinv)
    ridx = lax.broadcasted_iota(jnp.int32, (_NP, 1), 0)
    c = jnp.where(ridx < _N, c, 0.0)
    s = jnp.sum(c * a, axis=0, keepdims=True)
    out_ref[:, :] = (jnp.dot(s, w2_ref[:, :], preferred_element_type=jnp.float32)
                     * (1.0 / _N) + b2_ref[:, :])


def kernel(x, edge_index, edge_attr, batch, W1, b1, Wq, bq, Wk, bk,
           Wv, bv, Wo, bo, W2, b2):
    del batch, Wq, bq, Wk, bk
    row = edge_index[0]
    col = edge_index[1]
    row2 = row.reshape(_NW, _NCH, _CH)
    col2 = col.reshape(_NW, _NCH, _CH)
    ew2 = edge_attr.reshape(_NW, _EPT)
    xpad = jnp.pad(x, ((0, _NP - _N), (0, 0)))

    degp = _sc_deg(col2, ew2)

    xws, dinv2 = pl.pallas_call(
        _tc_pre_body,
        out_shape=(jax.ShapeDtypeStruct((_NP, _HID), jnp.float32),
                   jax.ShapeDtypeStruct((_NP, 1), jnp.float32)),
    )(xpad, W1, degp.T)

    dinv1 = dinv2[:_N, 0]
    feat, credp = _sc_agg(row2, col2, ew2, dinv1, xws)

    out = pl.pallas_call(
        _tc_post_body,
        out_shape=jax.ShapeDtypeStruct((1, 87), jnp.float32),
    )(feat, credp.T, xws, dinv2, b1[None], Wv, bv[None], Wo, bo[None],
      W2, b2[None])
    return out


# speedup vs baseline: 33.7543x; 1.0319x over previous
import functools

import jax
import jax.numpy as jnp
from jax import lax
from jax.experimental import pallas as pl
from jax.experimental.pallas import tpu as pltpu
from jax.experimental.pallas import tpu_sc as plsc

_N = 10000
_NP = 10240
_E = 320000
_NC = 2
_NS = 16
_NW = _NC * _NS
_EPT = _E // _NW
_MCH = 400
_NMC = _EPT // _MCH
_NPT = _NP // _NS
_HID = 64
_LANES = 16


def _mesh():
    return plsc.VectorSubcoreMesh(
        core_axis_name="c", subcore_axis_name="s",
        num_cores=_NC, num_subcores=_NS)


def _splat_lane(vec, lane):
    dn = lax.GatherDimensionNumbers(
        offset_dims=(), collapsed_slice_dims=(0,), start_index_map=(0,))
    idx = jnp.full((_LANES, 1), lane, jnp.int32)
    return lax.gather(vec, idx, dn, slice_sizes=(1,),
                      mode=lax.GatherScatterMode.PROMISE_IN_BOUNDS)


@functools.partial(
    pl.kernel,
    out_type=jax.ShapeDtypeStruct((_NC, _NP), jnp.float32),
    mesh=_mesh(),
    scratch_types=[
        pltpu.VMEM((_EPT,), jnp.int32),
        pltpu.VMEM((_EPT,), jnp.float32),
        pltpu.VMEM((_NPT,), jnp.float32),
        pltpu.VMEM_SHARED((_NP,), jnp.float32),
        pltpu.SemaphoreType.DMA,
    ],
)
def _sc_deg(col_hbm, ew_hbm, out_hbm, colv, ewv, zb, acc, sem):
    cid = lax.axis_index("c")
    sid = lax.axis_index("s")
    wid = cid * _NS + sid
    zero16 = jnp.zeros((_LANES,), jnp.float32)

    def zfill(i, _):
        zb[pl.ds(i * _LANES, _LANES)] = zero16
        return 0
    lax.fori_loop(0, _NPT // _LANES, zfill, 0)
    pltpu.sync_copy(zb, acc.at[pl.ds(sid * _NPT, _NPT)])
    plsc.subcore_barrier()

    pltpu.sync_copy(col_hbm.at[wid], colv)
    pltpu.sync_copy(ew_hbm.at[wid], ewv)

    _LAG = 4
    def scat(i):
        return pltpu.async_copy(
            ewv.at[pl.ds(i * _MCH, _MCH)],
            acc.at[colv.at[pl.ds(i * _MCH, _MCH)]], sem, add=True)
    for i in range(_NMC):
        scat(i)
        if i >= _LAG:
            pltpu.make_async_copy(
                ewv.at[pl.ds((i - _LAG) * _MCH, _MCH)],
                acc.at[colv.at[pl.ds((i - _LAG) * _MCH, _MCH)]], sem).wait()
    for j in range(_NMC - _LAG, _NMC):
        pltpu.make_async_copy(
            ewv.at[pl.ds(j * _MCH, _MCH)],
            acc.at[colv.at[pl.ds(j * _MCH, _MCH)]], sem).wait()

    plsc.subcore_barrier()
    pltpu.sync_copy(acc.at[pl.ds(sid * _NPT, _NPT)],
                    out_hbm.at[cid, pl.ds(sid * _NPT, _NPT)])


@functools.partial(
    pl.kernel,
    out_type=(jax.ShapeDtypeStruct((_NC, _NP, _HID), jnp.float32),
              jax.ShapeDtypeStruct((_NC, _NP), jnp.float32)),
    mesh=_mesh(),
    compiler_params=pltpu.CompilerParams(use_tc_tiling_on_sc=False),
    scratch_types=[
        pltpu.VMEM((_EPT,), jnp.int32),
        pltpu.VMEM((_EPT,), jnp.int32),
        pltpu.VMEM((_EPT,), jnp.float32),
        pltpu.VMEM((2, _MCH), jnp.float32),
        pltpu.VMEM((2, _MCH), jnp.float32),
        pltpu.VMEM((_MCH, _HID), jnp.float32),
        pltpu.VMEM((_MCH, _HID), jnp.float32),
        pltpu.VMEM((_NPT,), jnp.float32),
        pltpu.VMEM_SHARED((_NP, _HID), jnp.float32),
        pltpu.VMEM_SHARED((_NP,), jnp.float32),
        pltpu.SemaphoreType.DMA,
        pltpu.SemaphoreType.DMA,
        pltpu.SemaphoreType.DMA,
        pltpu.SemaphoreType.DMA,
        pltpu.SemaphoreType.DMA,
        pltpu.SemaphoreType.DMA,
        pltpu.SemaphoreType.DMA,
        pltpu.SemaphoreType.DMA,
    ],
)
def _sc_agg(row_hbm, col_hbm, ew_hbm, dinv_hbm, xws_hbm,
            feat_hbm, cred_hbm,
            rowv, colv, ewv, dval, cval, rows_a, rows_b, zb,
            spfeat, spcred,
            gsem_a, gsem_b, ssem_a, ssem_b, dsem_a, dsem_b, csem_a, csem_b):
    cid = lax.axis_index("c")
    sid = lax.axis_index("s")
    wid = cid * _NS + sid
    zero16 = jnp.zeros((_LANES,), jnp.float32)

    def zrow(i, _):
        for k in range(_HID // _LANES):
            rows_a[i, pl.ds(k * _LANES, _LANES)] = zero16
        return 0
    lax.fori_loop(0, _MCH, zrow, 0)
    for j in range(_NPT // _MCH + 1):
        n = min(_NPT - j * _MCH, _MCH)
        pltpu.sync_copy(rows_a.at[pl.ds(0, n)],
                        spfeat.at[pl.ds(sid * _NPT + j * _MCH, n)])

    def zfill(i, _):
        zb[pl.ds(i * _LANES, _LANES)] = zero16
        return 0
    lax.fori_loop(0, _NPT // _LANES, zfill, 0)
    pltpu.sync_copy(zb, spcred.at[pl.ds(sid * _NPT, _NPT)])
    plsc.subcore_barrier()

    pltpu.sync_copy(row_hbm.at[wid], rowv)
    pltpu.sync_copy(col_hbm.at[wid], colv)
    pltpu.sync_copy(ew_hbm.at[wid], ewv)

    bufs = (rows_a, rows_b)
    gsems = (gsem_a, gsem_b)
    ssems = (ssem_a, ssem_b)
    dsems = (dsem_a, dsem_b)
    csems = (csem_a, csem_b)

    def esl(i):
        return pl.ds(i * _MCH, _MCH)

    def issue_gather(i, p):
        pltpu.async_copy(xws_hbm.at[rowv.at[esl(i)]], bufs[p], gsems[p])
        pltpu.async_copy(dinv_hbm.at[colv.at[esl(i)]], dval.at[p], dsems[p])

    def wait_gather(i, p):
        pltpu.make_async_copy(xws_hbm.at[rowv.at[esl(i)]], bufs[p],
                              gsems[p]).wait()
        pltpu.make_async_copy(dinv_hbm.at[colv.at[esl(i)]], dval.at[p],
                              dsems[p]).wait()

    def scale(i, p):
        def grp(g, _):
            wv = ewv[pl.ds(i * _MCH + g * _LANES, _LANES)]
            dv = dval[p, pl.ds(g * _LANES, _LANES)]
            cval[p, pl.ds(g * _LANES, _LANES)] = wv * dv
            for e in range(_LANES):
                ws = _splat_lane(wv, e)
                r = g * _LANES + e
                for k in range(_HID // _LANES):
                    sl = pl.ds(k * _LANES, _LANES)
                    bufs[p][r, sl] = bufs[p][r, sl] * ws
            return 0
        lax.fori_loop(0, _MCH // _LANES, grp, 0)

    def issue_scatter(i, p):
        pltpu.async_copy(bufs[p], spfeat.at[colv.at[esl(i)]], ssems[p],
                         add=True)
        pltpu.async_copy(cval.at[p], spcred.at[rowv.at[esl(i)]], csems[p],
                         add=True)

    def wait_scatter(i, p):
        pltpu.make_async_copy(bufs[p], spfeat.at[colv.at[esl(i)]],
                              ssems[p]).wait()
        pltpu.make_async_copy(cval.at[p], spcred.at[rowv.at[esl(i)]],
                              csems[p]).wait()

    issue_gather(0, 0)

    def pair(t, _):
        i0 = 2 * t
        @pl.when(t > 0)
        def _():
            wait_scatter(i0 - 1, 1)
        issue_gather(i0 + 1, 1)
        wait_gather(i0, 0)
        scale(i0, 0)
        issue_scatter(i0, 0)
        wait_gather(i0 + 1, 1)
        scale(i0 + 1, 1)
        wait_scatter(i0, 0)
        issue_gather(i0 + 2, 0)
        issue_scatter(i0 + 1, 1)
        return 0
    lax.fori_loop(0, (_NMC - 1) // 2, pair, 0)

    last = _NMC - 1
    wait_scatter(last - 1, 1)
    wait_gather(last, 0)
    scale(last, 0)
    issue_scatter(last, 0)
    wait_scatter(last, 0)

    plsc.subcore_barrier()
    pltpu.sync_copy(spfeat.at[pl.ds(sid * _NPT, _NPT)],
                    feat_hbm.at[cid, pl.ds(sid * _NPT, _NPT)])
    pltpu.sync_copy(spcred.at[pl.ds(sid * _NPT, _NPT)],
                    cred_hbm.at[cid, pl.ds(sid * _NPT, _NPT)])


def _tc_pre_body(x_ref, w1_ref, degp_ref, xws_ref, dinv_ref):
    deg = degp_ref[:, 0:1] + degp_ref[:, 1:2] + 1.0
    dinv = lax.rsqrt(deg)
    xw = jnp.dot(x_ref[:, :], w1_ref[:, :], preferred_element_type=jnp.float32)
    xws_ref[:, :] = xw * dinv
    dinv_ref[:, :] = dinv


def _tc_post_body(feat_ref, credp_ref, xws_ref, dinv_ref, b1_ref,
                  wv_ref, bv_ref, wo_ref, bo_ref, w2_ref, b2_ref, out_ref):
    agg = feat_ref[0] + feat_ref[1]
    dinv = dinv_ref[:, :]
    h = jnp.maximum(dinv * (agg + xws_ref[:, :]) + b1_ref[:, :], 0.0)
    t = jnp.dot(h, wv_ref[:, :], preferred_element_type=jnp.float32) + bv_ref[:, :]
    a = jnp.dot(t, wo_ref[:, :], preferred_element_type=jnp.float32) + bo_ref[:, :]
    cred = credp_ref[:, 0:1] + credp_ref[:, 1:2]
    c = dinv * (cred + dinv)
    ridx = lax.broadcasted_iota(jnp.int32, (_NP, 1), 0)
    c = jnp.where(ridx < _N, c, 0.0)
    s = jnp.sum(c * a, axis=0, keepdims=True)
    out_ref[:, :] = (jnp.dot(s, w2_ref[:, :], preferred_element_type=jnp.float32)
                     * (1.0 / _N) + b2_ref[:, :])


def kernel(x, edge_index, edge_attr, batch, W1, b1, Wq, bq, Wk, bk,
           Wv, bv, Wo, bo, W2, b2):
    del batch, Wq, bq, Wk, bk
    row = edge_index[0]
    col = edge_index[1]
    roww = row.reshape(_NW, _EPT)
    colw = col.reshape(_NW, _EPT)
    ew2 = edge_attr.reshape(_NW, _EPT)
    xpad = jnp.pad(x, ((0, _NP - _N), (0, 0)))

    degp = _sc_deg(colw, ew2)

    xws, dinv2 = pl.pallas_call(
        _tc_pre_body,
        out_shape=(jax.ShapeDtypeStruct((_NP, _HID), jnp.float32),
                   jax.ShapeDtypeStruct((_NP, 1), jnp.float32)),
    )(xpad, W1, degp.T)

    dinv1 = dinv2[:_N, 0]
    feat, credp = _sc_agg(roww, colw, ew2, dinv1, xws)

    out = pl.pallas_call(
        _tc_post_body,
        out_shape=jax.ShapeDtypeStruct((1, 87), jnp.float32),
    )(feat, credp.T, xws, dinv2, b1[None], Wv, bv[None], Wo, bo[None],
      W2, b2[None])
    return out


# speedup vs baseline: 33.7780x; 1.0007x over previous
import functools

import jax
import jax.numpy as jnp
from jax import lax
from jax.experimental import pallas as pl
from jax.experimental.pallas import tpu as pltpu
from jax.experimental.pallas import tpu_sc as plsc

_N = 10000
_NP = 10240
_E = 320000
_NC = 2
_NS = 16
_NW = _NC * _NS
_EPT = _E // _NW
_MCH = 400
_NMC = _EPT // _MCH
_NPT = _NP // _NS
_HID = 64
_LANES = 16


def _mesh():
    return plsc.VectorSubcoreMesh(
        core_axis_name="c", subcore_axis_name="s",
        num_cores=_NC, num_subcores=_NS)


def _splat_lane(vec, lane):
    dn = lax.GatherDimensionNumbers(
        offset_dims=(), collapsed_slice_dims=(0,), start_index_map=(0,))
    idx = jnp.full((_LANES, 1), lane, jnp.int32)
    return lax.gather(vec, idx, dn, slice_sizes=(1,),
                      mode=lax.GatherScatterMode.PROMISE_IN_BOUNDS)


@functools.partial(
    pl.kernel,
    out_type=jax.ShapeDtypeStruct((_NC, _NP), jnp.float32),
    mesh=_mesh(),
    scratch_types=[
        pltpu.VMEM((_EPT,), jnp.int32),
        pltpu.VMEM((_EPT,), jnp.float32),
        pltpu.VMEM((_NPT,), jnp.float32),
        pltpu.VMEM_SHARED((_NP,), jnp.float32),
        pltpu.SemaphoreType.DMA,
    ],
)
def _sc_deg(col_hbm, ew_hbm, out_hbm, colv, ewv, zb, acc, sem):
    cid = lax.axis_index("c")
    sid = lax.axis_index("s")
    wid = cid * _NS + sid
    zero16 = jnp.zeros((_LANES,), jnp.float32)

    def zfill(i, _):
        zb[pl.ds(i * _LANES, _LANES)] = zero16
        return 0
    lax.fori_loop(0, _NPT // _LANES, zfill, 0)
    pltpu.sync_copy(zb, acc.at[pl.ds(sid * _NPT, _NPT)])
    plsc.subcore_barrier()

    pltpu.sync_copy(col_hbm.at[wid], colv)
    pltpu.sync_copy(ew_hbm.at[wid], ewv)

    _LAG = 4
    def scat(i):
        return pltpu.async_copy(
            ewv.at[pl.ds(i * _MCH, _MCH)],
            acc.at[colv.at[pl.ds(i * _MCH, _MCH)]], sem, add=True)
    for i in range(_NMC):
        scat(i)
        if i >= _LAG:
            pltpu.make_async_copy(
                ewv.at[pl.ds((i - _LAG) * _MCH, _MCH)],
                acc.at[colv.at[pl.ds((i - _LAG) * _MCH, _MCH)]], sem).wait()
    for j in range(_NMC - _LAG, _NMC):
        pltpu.make_async_copy(
            ewv.at[pl.ds(j * _MCH, _MCH)],
            acc.at[colv.at[pl.ds(j * _MCH, _MCH)]], sem).wait()

    plsc.subcore_barrier()
    pltpu.sync_copy(acc.at[pl.ds(sid * _NPT, _NPT)],
                    out_hbm.at[cid, pl.ds(sid * _NPT, _NPT)])


@functools.partial(
    pl.kernel,
    out_type=(jax.ShapeDtypeStruct((_NC, _NP, _HID), jnp.float32),
              jax.ShapeDtypeStruct((_NC, _NP), jnp.float32)),
    mesh=_mesh(),
    compiler_params=pltpu.CompilerParams(use_tc_tiling_on_sc=False),
    scratch_types=[
        pltpu.VMEM((_EPT,), jnp.int32),
        pltpu.VMEM((_EPT,), jnp.int32),
        pltpu.VMEM((_EPT,), jnp.float32),
        pltpu.VMEM((2, _MCH), jnp.float32),
        pltpu.VMEM((2, _MCH), jnp.float32),
        pltpu.VMEM((_MCH, _HID), jnp.float32),
        pltpu.VMEM((_MCH, _HID), jnp.float32),
        pltpu.VMEM((_NPT,), jnp.float32),
        pltpu.VMEM_SHARED((_NP, _HID), jnp.float32),
        pltpu.VMEM_SHARED((_NP,), jnp.float32),
        pltpu.SemaphoreType.DMA,
        pltpu.SemaphoreType.DMA,
        pltpu.SemaphoreType.DMA,
        pltpu.SemaphoreType.DMA,
        pltpu.SemaphoreType.DMA,
        pltpu.SemaphoreType.DMA,
        pltpu.SemaphoreType.DMA,
        pltpu.SemaphoreType.DMA,
    ],
)
def _sc_agg(row_hbm, col_hbm, ew_hbm, dinv_hbm, xws_hbm,
            feat_hbm, cred_hbm,
            rowv, colv, ewv, dval, cval, rows_a, rows_b, zb,
            spfeat, spcred,
            gsem_a, gsem_b, ssem_a, ssem_b, dsem_a, dsem_b, csem_a, csem_b):
    cid = lax.axis_index("c")
    sid = lax.axis_index("s")
    wid = cid * _NS + sid
    zero16 = jnp.zeros((_LANES,), jnp.float32)

    def zrow(i, _):
        for k in range(_HID // _LANES):
            rows_a[i, pl.ds(k * _LANES, _LANES)] = zero16
        return 0
    lax.fori_loop(0, _MCH, zrow, 0)
    for j in range(_NPT // _MCH + 1):
        n = min(_NPT - j * _MCH, _MCH)
        pltpu.sync_copy(rows_a.at[pl.ds(0, n)],
                        spfeat.at[pl.ds(sid * _NPT + j * _MCH, n)])

    def zfill(i, _):
        zb[pl.ds(i * _LANES, _LANES)] = zero16
        return 0
    lax.fori_loop(0, _NPT // _LANES, zfill, 0)
    pltpu.sync_copy(zb, spcred.at[pl.ds(sid * _NPT, _NPT)])
    plsc.subcore_barrier()

    pltpu.sync_copy(row_hbm.at[wid], rowv)
    pltpu.sync_copy(col_hbm.at[wid], colv)
    pltpu.sync_copy(ew_hbm.at[wid], ewv)

    bufs = (rows_a, rows_b)
    gsems = (gsem_a, gsem_b)
    ssems = (ssem_a, ssem_b)
    dsems = (dsem_a, dsem_b)
    csems = (csem_a, csem_b)

    def esl(i):
        return pl.ds(i * _MCH, _MCH)

    def issue_gather(i, p):
        pltpu.async_copy(xws_hbm.at[rowv.at[esl(i)]], bufs[p], gsems[p])
        pltpu.async_copy(dinv_hbm.at[colv.at[esl(i)]], dval.at[p], dsems[p])

    def wait_gather(i, p):
        pltpu.make_async_copy(xws_hbm.at[rowv.at[esl(i)]], bufs[p],
                              gsems[p]).wait()
        pltpu.make_async_copy(dinv_hbm.at[colv.at[esl(i)]], dval.at[p],
                              dsems[p]).wait()

    def scale(i, p):
        def grp(g, _):
            wv = ewv[pl.ds(i * _MCH + g * _LANES, _LANES)]
            dv = dval[p, pl.ds(g * _LANES, _LANES)]
            cval[p, pl.ds(g * _LANES, _LANES)] = wv * dv
            sub = bufs[p].at[pl.ds(g * _LANES, _LANES)]
            for e in range(_LANES):
                ws = _splat_lane(wv, e)
                for k in range(_HID // _LANES):
                    sl = pl.ds(k * _LANES, _LANES)
                    sub[e, sl] = sub[e, sl] * ws
            return 0
        lax.fori_loop(0, _MCH // _LANES, grp, 0)

    def issue_scatter(i, p):
        pltpu.async_copy(bufs[p], spfeat.at[colv.at[esl(i)]], ssems[p],
                         add=True)
        pltpu.async_copy(cval.at[p], spcred.at[rowv.at[esl(i)]], csems[p],
                         add=True)

    def wait_scatter(i, p):
        pltpu.make_async_copy(bufs[p], spfeat.at[colv.at[esl(i)]],
                              ssems[p]).wait()
        pltpu.make_async_copy(cval.at[p], spcred.at[rowv.at[esl(i)]],
                              csems[p]).wait()

    issue_gather(0, 0)

    def pair(t, _):
        i0 = 2 * t
        @pl.when(t > 0)
        def _():
            wait_scatter(i0 - 1, 1)
        issue_gather(i0 + 1, 1)
        wait_gather(i0, 0)
        scale(i0, 0)
        issue_scatter(i0, 0)
        wait_gather(i0 + 1, 1)
        scale(i0 + 1, 1)
        wait_scatter(i0, 0)
        issue_gather(i0 + 2, 0)
        issue_scatter(i0 + 1, 1)
        return 0
    lax.fori_loop(0, (_NMC - 1) // 2, pair, 0)

    last = _NMC - 1
    wait_scatter(last - 1, 1)
    wait_gather(last, 0)
    scale(last, 0)
    issue_scatter(last, 0)
    wait_scatter(last, 0)

    plsc.subcore_barrier()
    pltpu.sync_copy(spfeat.at[pl.ds(sid * _NPT, _NPT)],
                    feat_hbm.at[cid, pl.ds(sid * _NPT, _NPT)])
    pltpu.sync_copy(spcred.at[pl.ds(sid * _NPT, _NPT)],
                    cred_hbm.at[cid, pl.ds(sid * _NPT, _NPT)])


def _tc_pre_body(x_ref, w1_ref, degp_ref, xws_ref, dinv_ref):
    deg = degp_ref[:, 0:1] + degp_ref[:, 1:2] + 1.0
    dinv = lax.rsqrt(deg)
    xw = jnp.dot(x_ref[:, :], w1_ref[:, :], preferred_element_type=jnp.float32)
    xws_ref[:, :] = xw * dinv
    dinv_ref[:, :] = dinv


def _tc_post_body(feat_ref, credp_ref, xws_ref, dinv_ref, b1_ref,
                  wv_ref, bv_ref, wo_ref, bo_ref, w2_ref, b2_ref, out_ref):
    agg = feat_ref[0] + feat_ref[1]
    dinv = dinv_ref[:, :]
    h = jnp.maximum(dinv * (agg + xws_ref[:, :]) + b1_ref[:, :], 0.0)
    t = jnp.dot(h, wv_ref[:, :], preferred_element_type=jnp.float32) + bv_ref[:, :]
    a = jnp.dot(t, wo_ref[:, :], preferred_element_type=jnp.float32) + bo_ref[:, :]
    cred = credp_ref[:, 0:1] + credp_ref[:, 1:2]
    c = dinv * (cred + dinv)
    ridx = lax.broadcasted_iota(jnp.int32, (_NP, 1), 0)
    c = jnp.where(ridx < _N, c, 0.0)
    s = jnp.sum(c * a, axis=0, keepdims=True)
    out_ref[:, :] = (jnp.dot(s, w2_ref[:, :], preferred_element_type=jnp.float32)
                     * (1.0 / _N) + b2_ref[:, :])


def kernel(x, edge_index, edge_attr, batch, W1, b1, Wq, bq, Wk, bk,
           Wv, bv, Wo, bo, W2, b2):
    del batch, Wq, bq, Wk, bk
    row = edge_index[0]
    col = edge_index[1]
    roww = row.reshape(_NW, _EPT)
    colw = col.reshape(_NW, _EPT)
    ew2 = edge_attr.reshape(_NW, _EPT)
    xpad = jnp.pad(x, ((0, _NP - _N), (0, 0)))

    degp = _sc_deg(colw, ew2)

    xws, dinv2 = pl.pallas_call(
        _tc_pre_body,
        out_shape=(jax.ShapeDtypeStruct((_NP, _HID), jnp.float32),
                   jax.ShapeDtypeStruct((_NP, 1), jnp.float32)),
    )(xpad, W1, degp.T)

    dinv1 = dinv2[:_N, 0]
    feat, credp = _sc_agg(roww, colw, ew2, dinv1, xws)

    out = pl.pallas_call(
        _tc_post_body,
        out_shape=jax.ShapeDtypeStruct((1, 87), jnp.float32),
    )(feat, credp.T, xws, dinv2, b1[None], Wv, bv[None], Wo, bo[None],
      W2, b2[None])
    return out


# speedup vs baseline: 56.2702x; 1.6659x over previous
import functools

import jax
import jax.numpy as jnp
from jax import lax
from jax.experimental import pallas as pl
from jax.experimental.pallas import tpu as pltpu
from jax.experimental.pallas import tpu_sc as plsc

_N = 10000
_NP = 10240
_E = 320000
_NC = 2
_NS = 16
_NW = _NC * _NS
_EPT = _E // _NW
_MCH = 400
_NMC = _EPT // _MCH
_NPT = _NP // _NS
_HID = 64
_LANES = 16


def _mesh():
    return plsc.VectorSubcoreMesh(
        core_axis_name="c", subcore_axis_name="s",
        num_cores=_NC, num_subcores=_NS)


def _splat_lane(vec, lane):
    dn = lax.GatherDimensionNumbers(
        offset_dims=(), collapsed_slice_dims=(0,), start_index_map=(0,))
    idx = jnp.full((_LANES, 1), lane, jnp.int32)
    return lax.gather(vec, idx, dn, slice_sizes=(1,),
                      mode=lax.GatherScatterMode.PROMISE_IN_BOUNDS)


@functools.partial(
    pl.kernel,
    out_type=jax.ShapeDtypeStruct((_NC, _NP), jnp.float32),
    mesh=_mesh(),
    scratch_types=[
        pltpu.VMEM((_EPT,), jnp.int32),
        pltpu.VMEM((_EPT,), jnp.float32),
        pltpu.VMEM((_NPT,), jnp.float32),
        pltpu.VMEM_SHARED((_NP,), jnp.float32),
        pltpu.SemaphoreType.DMA,
    ],
)
def _sc_deg(col_hbm, ew_hbm, out_hbm, colv, ewv, zb, acc, sem):
    cid = lax.axis_index("c")
    sid = lax.axis_index("s")
    wid = cid * _NS + sid
    zero16 = jnp.zeros((_LANES,), jnp.float32)

    def zfill(i, _):
        zb[pl.ds(i * _LANES, _LANES)] = zero16
        return 0
    lax.fori_loop(0, _NPT // _LANES, zfill, 0)
    pltpu.sync_copy(zb, acc.at[pl.ds(sid * _NPT, _NPT)])
    plsc.subcore_barrier()

    pltpu.sync_copy(col_hbm.at[wid], colv)
    pltpu.sync_copy(ew_hbm.at[wid], ewv)

    _LAG = 4
    def scat(i):
        return pltpu.async_copy(
            ewv.at[pl.ds(i * _MCH, _MCH)],
            acc.at[colv.at[pl.ds(i * _MCH, _MCH)]], sem, add=True)
    for i in range(_NMC):
        scat(i)
        if i >= _LAG:
            pltpu.make_async_copy(
                ewv.at[pl.ds((i - _LAG) * _MCH, _MCH)],
                acc.at[colv.at[pl.ds((i - _LAG) * _MCH, _MCH)]], sem).wait()
    for j in range(_NMC - _LAG, _NMC):
        pltpu.make_async_copy(
            ewv.at[pl.ds(j * _MCH, _MCH)],
            acc.at[colv.at[pl.ds(j * _MCH, _MCH)]], sem).wait()

    plsc.subcore_barrier()
    pltpu.sync_copy(acc.at[pl.ds(sid * _NPT, _NPT)],
                    out_hbm.at[cid, pl.ds(sid * _NPT, _NPT)])


@functools.partial(
    pl.kernel,
    out_type=(jax.ShapeDtypeStruct((_NC, _NP, _HID), jnp.float32),
              jax.ShapeDtypeStruct((_NC, _NP), jnp.float32)),
    mesh=_mesh(),
    compiler_params=pltpu.CompilerParams(use_tc_tiling_on_sc=False),
    scratch_types=[
        pltpu.VMEM((_EPT,), jnp.int32),
        pltpu.VMEM((_EPT,), jnp.int32),
        pltpu.VMEM((_EPT,), jnp.float32),
        pltpu.VMEM((2, _MCH), jnp.float32),
        pltpu.VMEM((2, _MCH), jnp.float32),
        pltpu.VMEM((_MCH, _HID), jnp.float32),
        pltpu.VMEM((_MCH, _HID), jnp.float32),
        pltpu.VMEM((_NPT,), jnp.float32),
        pltpu.VMEM_SHARED((_NP, _HID), jnp.float32),
        pltpu.VMEM_SHARED((_NP,), jnp.float32),
        pltpu.SemaphoreType.DMA,
        pltpu.SemaphoreType.DMA,
        pltpu.SemaphoreType.DMA,
        pltpu.SemaphoreType.DMA,
        pltpu.SemaphoreType.DMA,
        pltpu.SemaphoreType.DMA,
        pltpu.SemaphoreType.DMA,
        pltpu.SemaphoreType.DMA,
    ],
)
def _sc_agg(row_hbm, col_hbm, ew_hbm, dinv_hbm, xws_hbm,
            feat_hbm, cred_hbm,
            rowv, colv, ewv, dval, cval, rows_a, rows_b, zb,
            spfeat, spcred,
            gsem_a, gsem_b, ssem_a, ssem_b, dsem_a, dsem_b, csem_a, csem_b):
    cid = lax.axis_index("c")
    sid = lax.axis_index("s")
    wid = cid * _NS + sid
    zero16 = jnp.zeros((_LANES,), jnp.float32)

    def zrow(i, _):
        for k in range(_HID // _LANES):
            rows_a[i, pl.ds(k * _LANES, _LANES)] = zero16
        return 0
    lax.fori_loop(0, _MCH, zrow, 0)
    for j in range(_NPT // _MCH + 1):
        n = min(_NPT - j * _MCH, _MCH)
        pltpu.sync_copy(rows_a.at[pl.ds(0, n)],
                        spfeat.at[pl.ds(sid * _NPT + j * _MCH, n)])

    def zfill(i, _):
        zb[pl.ds(i * _LANES, _LANES)] = zero16
        return 0
    lax.fori_loop(0, _NPT // _LANES, zfill, 0)
    pltpu.sync_copy(zb, spcred.at[pl.ds(sid * _NPT, _NPT)])
    plsc.subcore_barrier()

    pltpu.sync_copy(row_hbm.at[wid], rowv)
    pltpu.sync_copy(col_hbm.at[wid], colv)
    pltpu.sync_copy(ew_hbm.at[wid], ewv)

    bufs = (rows_a, rows_b)
    gsems = (gsem_a, gsem_b)
    ssems = (ssem_a, ssem_b)
    dsems = (dsem_a, dsem_b)
    csems = (csem_a, csem_b)

    def esl(i):
        return pl.ds(i * _MCH, _MCH)

    def issue_gather(i, p):
        pltpu.async_copy(xws_hbm.at[rowv.at[esl(i)]], bufs[p], gsems[p])
        pltpu.async_copy(dinv_hbm.at[colv.at[esl(i)]], dval.at[p], dsems[p])

    def wait_gather(i, p):
        pltpu.make_async_copy(xws_hbm.at[rowv.at[esl(i)]], bufs[p],
                              gsems[p]).wait()
        pltpu.make_async_copy(dinv_hbm.at[colv.at[esl(i)]], dval.at[p],
                              dsems[p]).wait()

    def scale(i, p):
        @functools.partial(plsc.parallel_loop, 0, _MCH // _LANES, unroll=2)
        def grp(g):
            wv = ewv[pl.ds(i * _MCH + g * _LANES, _LANES)]
            dv = dval[p, pl.ds(g * _LANES, _LANES)]
            cval[p, pl.ds(g * _LANES, _LANES)] = wv * dv
            sub = bufs[p].at[pl.ds(g * _LANES, _LANES)]
            nk = _HID // _LANES
            for e in range(_LANES):
                ws = _splat_lane(wv, e)
                vals = [sub[e, pl.ds(k * _LANES, _LANES)] for k in range(nk)]
                for k in range(nk):
                    sub[e, pl.ds(k * _LANES, _LANES)] = vals[k] * ws

    def issue_scatter(i, p):
        pltpu.async_copy(bufs[p], spfeat.at[colv.at[esl(i)]], ssems[p],
                         add=True)
        pltpu.async_copy(cval.at[p], spcred.at[rowv.at[esl(i)]], csems[p],
                         add=True)

    def wait_scatter(i, p):
        pltpu.make_async_copy(bufs[p], spfeat.at[colv.at[esl(i)]],
                              ssems[p]).wait()
        pltpu.make_async_copy(cval.at[p], spcred.at[rowv.at[esl(i)]],
                              csems[p]).wait()

    issue_gather(0, 0)

    def pair(t, _):
        i0 = 2 * t
        @pl.when(t > 0)
        def _():
            wait_scatter(i0 - 1, 1)
        issue_gather(i0 + 1, 1)
        wait_gather(i0, 0)
        scale(i0, 0)
        issue_scatter(i0, 0)
        wait_gather(i0 + 1, 1)
        scale(i0 + 1, 1)
        wait_scatter(i0, 0)
        issue_gather(i0 + 2, 0)
        issue_scatter(i0 + 1, 1)
        return 0
    lax.fori_loop(0, (_NMC - 1) // 2, pair, 0)

    last = _NMC - 1
    wait_scatter(last - 1, 1)
    wait_gather(last, 0)
    scale(last, 0)
    issue_scatter(last, 0)
    wait_scatter(last, 0)

    plsc.subcore_barrier()
    pltpu.sync_copy(spfeat.at[pl.ds(sid * _NPT, _NPT)],
                    feat_hbm.at[cid, pl.ds(sid * _NPT, _NPT)])
    pltpu.sync_copy(spcred.at[pl.ds(sid * _NPT, _NPT)],
                    cred_hbm.at[cid, pl.ds(sid * _NPT, _NPT)])


def _tc_pre_body(x_ref, w1_ref, degp_ref, xws_ref, dinv_ref):
    deg = degp_ref[:, 0:1] + degp_ref[:, 1:2] + 1.0
    dinv = lax.rsqrt(deg)
    xw = jnp.dot(x_ref[:, :], w1_ref[:, :], preferred_element_type=jnp.float32)
    xws_ref[:, :] = xw * dinv
    dinv_ref[:, :] = dinv


def _tc_post_body(feat_ref, credp_ref, xws_ref, dinv_ref, b1_ref,
                  wv_ref, bv_ref, wo_ref, bo_ref, w2_ref, b2_ref, out_ref):
    agg = feat_ref[0] + feat_ref[1]
    dinv = dinv_ref[:, :]
    h = jnp.maximum(dinv * (agg + xws_ref[:, :]) + b1_ref[:, :], 0.0)
    t = jnp.dot(h, wv_ref[:, :], preferred_element_type=jnp.float32) + bv_ref[:, :]
    a = jnp.dot(t, wo_ref[:, :], preferred_element_type=jnp.float32) + bo_ref[:, :]
    cred = credp_ref[:, 0:1] + credp_ref[:, 1:2]
    c = dinv * (cred + dinv)
    ridx = lax.broadcasted_iota(jnp.int32, (_NP, 1), 0)
    c = jnp.where(ridx < _N, c, 0.0)
    s = jnp.sum(c * a, axis=0, keepdims=True)
    out_ref[:, :] = (jnp.dot(s, w2_ref[:, :], preferred_element_type=jnp.float32)
                     * (1.0 / _N) + b2_ref[:, :])


def kernel(x, edge_index, edge_attr, batch, W1, b1, Wq, bq, Wk, bk,
           Wv, bv, Wo, bo, W2, b2):
    del batch, Wq, bq, Wk, bk
    row = edge_index[0]
    col = edge_index[1]
    roww = row.reshape(_NW, _EPT)
    colw = col.reshape(_NW, _EPT)
    ew2 = edge_attr.reshape(_NW, _EPT)
    xpad = jnp.pad(x, ((0, _NP - _N), (0, 0)))

    degp = _sc_deg(colw, ew2)

    xws, dinv2 = pl.pallas_call(
        _tc_pre_body,
        out_shape=(jax.ShapeDtypeStruct((_NP, _HID), jnp.float32),
                   jax.ShapeDtypeStruct((_NP, 1), jnp.float32)),
    )(xpad, W1, degp.T)

    dinv1 = dinv2[:_N, 0]
    feat, credp = _sc_agg(roww, colw, ew2, dinv1, xws)

    out = pl.pallas_call(
        _tc_post_body,
        out_shape=jax.ShapeDtypeStruct((1, 87), jnp.float32),
    )(feat, credp.T, xws, dinv2, b1[None], Wv, bv[None], Wo, bo[None],
      W2, b2[None])
    return out
